# Initial kernel scaffold; baseline (speedup 1.0000x reference)
import jax
import jax.numpy as jnp
from jax.experimental import pallas as pl


def kernel(x, edge_index, batch, W1_0, b1_0, W2_0, b2_0, g_0, be_0, W1_1, b1_1, W2_1, b2_1, g_1, be_1, W1_2, b1_2, W2_2, b2_2, g_2, be_2, Wp1, bp1, Wp2, bp2):
    raise NotImplementedError("write your pallas kernel here")


# speedup vs baseline: 6.4158x; 6.4158x over previous
import functools

import jax
import jax.numpy as jnp
from jax import lax
from jax.experimental import pallas as pl
from jax.experimental.pallas import tpu as pltpu
from jax.experimental.pallas import tpu_sc as plsc

_N = 10000
_E = 320000
_D = 128
_G = 128
_NC = 2
_NS = 16
_NW = _NC * _NS
_EPT = _E // _NW
_K = 80
_NCH = _EPT // _K
_NP = 10240
_RPT = _NP // _NS


def _seg_sum_sc(h, src, dst):
    mesh = plsc.VectorSubcoreMesh(
        core_axis_name="c", subcore_axis_name="s",
        num_cores=_NC, num_subcores=_NS)

    @functools.partial(
        pl.kernel, mesh=mesh,
        out_type=jax.ShapeDtypeStruct((_NC, _NP, _D), jnp.float32),
        scratch_types=[
            pltpu.VMEM((_EPT,), jnp.int32),
            pltpu.VMEM((_NCH, _K), jnp.int32),
            pltpu.VMEM((_K, _D), jnp.float32),
            pltpu.VMEM_SHARED((_NP, _D), jnp.float32),
            pltpu.SemaphoreType.DMA,
        ],
    )
    def k(h_hbm, src_hbm, dst_hbm, out_hbm, src_v, dst_v, rows_v,
          acc_sh, sem):
        cid = lax.axis_index("c")
        sid = lax.axis_index("s")
        wid = sid * _NC + cid

        def zrow(i, _):
            def zcol(j, _):
                rows_v[i, pl.ds(j * 16, 16)] = jnp.zeros((16,), jnp.float32)
                return 0
            return lax.fori_loop(0, _D // 16, zcol, 0)
        lax.fori_loop(0, _K, zrow, 0)

        def zacc(i, _):
            pltpu.sync_copy(
                rows_v, acc_sh.at[pl.ds(sid * _RPT + i * _K, _K)])
            return 0
        lax.fori_loop(0, _RPT // _K, zacc, 0)

        pltpu.sync_copy(src_hbm.at[wid], src_v)
        pltpu.sync_copy(dst_hbm.at[wid], dst_v)
        plsc.subcore_barrier()

        def body(j, _):
            sl = pl.ds(pl.multiple_of(j * _K, _K), _K)
            pltpu.async_copy(h_hbm.at[src_v.at[sl]], rows_v, sem).wait()
            pltpu.sync_copy(rows_v, acc_sh.at[dst_v.at[j]], add=True)
            return 0
        lax.fori_loop(0, _NCH, body, 0)
        plsc.subcore_barrier()

        pltpu.sync_copy(acc_sh.at[pl.ds(sid * _RPT, _RPT)],
                        out_hbm.at[cid, pl.ds(sid * _RPT, _RPT)])

    return k(h, src, dst)


def _layer_tc(h, agg2, w1, b1, w2, b2, g, be):
    def body(h_ref, a_ref, w1_ref, b1_ref, w2_ref, b2_ref, g_ref, be_ref,
             o_ref):
        h2 = h_ref[...] + a_ref[0, :_N] + a_ref[1, :_N]
        z = jnp.dot(h2, w1_ref[...], preferred_element_type=jnp.float32)
        z = jnp.maximum(z + b1_ref[...], 0.0)
        z = jnp.dot(z, w2_ref[...], preferred_element_type=jnp.float32)
        z = jnp.maximum(z + b2_ref[...], 0.0)
        m = jnp.mean(z, axis=0, keepdims=True)
        c = z - m
        v = jnp.mean(c * c, axis=0, keepdims=True)
        o_ref[...] = c * lax.rsqrt(v + 1e-5) * g_ref[...] + be_ref[...]

    return pl.pallas_call(
        body, out_shape=jax.ShapeDtypeStruct((_N, _D), jnp.float32),
    )(h, agg2, w1, b1.reshape(1, _D), w2, b2.reshape(1, _D),
      g.reshape(1, _D), be.reshape(1, _D))


def _head_tc(h1, h2, h3, b_row, wp1, bp1, wp2, bp2):
    def body(h1_ref, h2_ref, h3_ref, b_ref, wp1_ref, bp1_ref, wp2_ref,
             bp2_ref, y_ref, xc_ref):
        gid = lax.broadcasted_iota(jnp.int32, (_G, _N), 0)
        oht = (b_ref[...] == gid).astype(jnp.float32)
        p1 = jnp.dot(oht, h1_ref[...], preferred_element_type=jnp.float32)
        p2 = jnp.dot(oht, h2_ref[...], preferred_element_type=jnp.float32)
        p3 = jnp.dot(oht, h3_ref[...], preferred_element_type=jnp.float32)
        xc = jnp.concatenate([p1, p2, p3], axis=1)
        y = jnp.dot(xc, wp1_ref[...], preferred_element_type=jnp.float32)
        y = jnp.maximum(y + bp1_ref[...], 0.0)
        y = jnp.dot(y, wp2_ref[...], preferred_element_type=jnp.float32)
        y = y + bp2_ref[...]
        yn = jnp.sqrt(jnp.sum(y * y, axis=1, keepdims=True))
        y_ref[...] = y / jnp.maximum(yn, 1e-12)
        xn = jnp.sqrt(jnp.sum(xc * xc, axis=1, keepdims=True))
        xc_ref[...] = xc / jnp.maximum(xn, 1e-12)

    return pl.pallas_call(
        body,
        out_shape=(jax.ShapeDtypeStruct((_G, 3 * _D), jnp.float32),
                   jax.ShapeDtypeStruct((_G, 3 * _D), jnp.float32)),
    )(h1, h2, h3, b_row, wp1, bp1.reshape(1, 3 * _D), wp2,
      bp2.reshape(1, 3 * _D))


def kernel(x, edge_index, batch,
           W1_0, b1_0, W2_0, b2_0, g_0, be_0,
           W1_1, b1_1, W2_1, b2_1, g_1, be_1,
           W1_2, b1_2, W2_2, b2_2, g_2, be_2,
           Wp1, bp1, Wp2, bp2):
    src = edge_index[0].astype(jnp.int32).reshape(_NW, _EPT)
    dst = edge_index[1].astype(jnp.int32).reshape(_NW, _NCH, _K)
    b_row = batch.astype(jnp.int32).reshape(1, _N)

    params = [
        (W1_0, b1_0, W2_0, b2_0, g_0, be_0),
        (W1_1, b1_1, W2_1, b2_1, g_1, be_1),
        (W1_2, b1_2, W2_2, b2_2, g_2, be_2),
    ]
    h = x
    hs = []
    for (w1, b1, w2, b2, g, be) in params:
        agg2 = _seg_sum_sc(h, src, dst)
        h = _layer_tc(h, agg2, w1, b1, w2, b2, g, be)
        hs.append(h)
    return _head_tc(hs[0], hs[1], hs[2], b_row, Wp1, bp1, Wp2, bp2)


# speedup vs baseline: 10.1120x; 1.5761x over previous
import functools

import jax
import jax.numpy as jnp
from jax import lax
from jax.experimental import pallas as pl
from jax.experimental.pallas import tpu as pltpu
from jax.experimental.pallas import tpu_sc as plsc

_N = 10000
_E = 320000
_D = 128
_G = 128
_NC = 2
_NS = 16
_NW = _NC * _NS
_EPT = _E // _NW
_K = 80
_NCH = _EPT // _K
_NP = 10240
_RPT = _NP // _NS


def _seg_sum_sc(h, src, dst):
    mesh = plsc.VectorSubcoreMesh(
        core_axis_name="c", subcore_axis_name="s",
        num_cores=_NC, num_subcores=_NS)

    @functools.partial(
        pl.kernel, mesh=mesh,
        out_type=jax.ShapeDtypeStruct((_NC, _NP, _D), jnp.float32),
        scratch_types=[
            pltpu.VMEM((_EPT,), jnp.int32),
            pltpu.VMEM((_NCH, _K), jnp.int32),
            pltpu.VMEM((_K, _D), jnp.float32),
            pltpu.VMEM((_K, _D), jnp.float32),
            pltpu.VMEM_SHARED((_NP, _D), jnp.float32),
            pltpu.SemaphoreType.DMA,
            pltpu.SemaphoreType.DMA,
        ],
    )
    def k(h_hbm, src_hbm, dst_hbm, out_hbm, src_v, dst_v, rows0, rows1,
          acc_sh, sem_g, sem_s):
        cid = lax.axis_index("c")
        sid = lax.axis_index("s")
        wid = sid * _NC + cid

        def zrow(i, _):
            def zcol(j, _):
                rows0[i, pl.ds(j * 16, 16)] = jnp.zeros((16,), jnp.float32)
                return 0
            return lax.fori_loop(0, _D // 16, zcol, 0)
        lax.fori_loop(0, _K, zrow, 0)

        def zacc(i, _):
            pltpu.sync_copy(
                rows0, acc_sh.at[pl.ds(sid * _RPT + i * _K, _K)])
            return 0
        lax.fori_loop(0, _RPT // _K, zacc, 0)

        pltpu.sync_copy(src_hbm.at[wid], src_v)
        pltpu.sync_copy(dst_hbm.at[wid], dst_v)
        plsc.subcore_barrier()

        def g_issue(j, buf):
            sl = pl.ds(pl.multiple_of(jnp.minimum(j, _NCH - 1) * _K, _K), _K)
            pltpu.async_copy(h_hbm.at[src_v.at[sl]], buf, sem_g)

        def g_wait(buf):
            pltpu.make_async_copy(
                h_hbm.at[src_v.at[pl.ds(0, _K)]], buf, sem_g).wait()

        def s_issue(j, buf):
            pltpu.async_copy(buf, acc_sh.at[dst_v.at[j]], sem_s, add=True)

        def s_wait(j, buf):
            pltpu.make_async_copy(
                buf, acc_sh.at[dst_v.at[j]], sem_s).wait()

        g_issue(0, rows0)
        g_issue(1, rows1)

        def body(i, _):
            j0 = i * 2
            g_wait(rows0)
            s_issue(j0, rows0)
            g_wait(rows1)
            s_issue(j0 + 1, rows1)
            s_wait(j0, rows0)
            g_issue(j0 + 2, rows0)
            s_wait(j0 + 1, rows1)
            g_issue(j0 + 3, rows1)
            return 0
        lax.fori_loop(0, _NCH // 2, body, 0)

        g_wait(rows0)
        pltpu.sync_copy(rows0, acc_sh.at[dst_v.at[_NCH - 1]], add=True)
        g_wait(rows1)
        plsc.subcore_barrier()

        pltpu.sync_copy(acc_sh.at[pl.ds(sid * _RPT, _RPT)],
                        out_hbm.at[cid, pl.ds(sid * _RPT, _RPT)])

    return k(h, src, dst)


def _layer_tc(h, agg2, w1, b1, w2, b2, g, be):
    def body(h_ref, a_ref, w1_ref, b1_ref, w2_ref, b2_ref, g_ref, be_ref,
             o_ref):
        h2 = h_ref[...] + a_ref[0, :_N] + a_ref[1, :_N]
        z = jnp.dot(h2, w1_ref[...], preferred_element_type=jnp.float32)
        z = jnp.maximum(z + b1_ref[...], 0.0)
        z = jnp.dot(z, w2_ref[...], preferred_element_type=jnp.float32)
        z = jnp.maximum(z + b2_ref[...], 0.0)
        m = jnp.mean(z, axis=0, keepdims=True)
        c = z - m
        v = jnp.mean(c * c, axis=0, keepdims=True)
        o_ref[...] = c * lax.rsqrt(v + 1e-5) * g_ref[...] + be_ref[...]

    return pl.pallas_call(
        body, out_shape=jax.ShapeDtypeStruct((_N, _D), jnp.float32),
    )(h, agg2, w1, b1.reshape(1, _D), w2, b2.reshape(1, _D),
      g.reshape(1, _D), be.reshape(1, _D))


def _head_tc(h1, h2, h3, b_row, wp1, bp1, wp2, bp2):
    def body(h1_ref, h2_ref, h3_ref, b_ref, wp1_ref, bp1_ref, wp2_ref,
             bp2_ref, y_ref, xc_ref):
        gid = lax.broadcasted_iota(jnp.int32, (_G, _N), 0)
        oht = (b_ref[...] == gid).astype(jnp.float32)
        p1 = jnp.dot(oht, h1_ref[...], preferred_element_type=jnp.float32)
        p2 = jnp.dot(oht, h2_ref[...], preferred_element_type=jnp.float32)
        p3 = jnp.dot(oht, h3_ref[...], preferred_element_type=jnp.float32)
        xc = jnp.concatenate([p1, p2, p3], axis=1)
        y = jnp.dot(xc, wp1_ref[...], preferred_element_type=jnp.float32)
        y = jnp.maximum(y + bp1_ref[...], 0.0)
        y = jnp.dot(y, wp2_ref[...], preferred_element_type=jnp.float32)
        y = y + bp2_ref[...]
        yn = jnp.sqrt(jnp.sum(y * y, axis=1, keepdims=True))
        y_ref[...] = y / jnp.maximum(yn, 1e-12)
        xn = jnp.sqrt(jnp.sum(xc * xc, axis=1, keepdims=True))
        xc_ref[...] = xc / jnp.maximum(xn, 1e-12)

    return pl.pallas_call(
        body,
        out_shape=(jax.ShapeDtypeStruct((_G, 3 * _D), jnp.float32),
                   jax.ShapeDtypeStruct((_G, 3 * _D), jnp.float32)),
    )(h1, h2, h3, b_row, wp1, bp1.reshape(1, 3 * _D), wp2,
      bp2.reshape(1, 3 * _D))


def kernel(x, edge_index, batch,
           W1_0, b1_0, W2_0, b2_0, g_0, be_0,
           W1_1, b1_1, W2_1, b2_1, g_1, be_1,
           W1_2, b1_2, W2_2, b2_2, g_2, be_2,
           Wp1, bp1, Wp2, bp2):
    src = edge_index[0].astype(jnp.int32).reshape(_NW, _EPT)
    dst = edge_index[1].astype(jnp.int32).reshape(_NW, _NCH, _K)
    b_row = batch.astype(jnp.int32).reshape(1, _N)

    params = [
        (W1_0, b1_0, W2_0, b2_0, g_0, be_0),
        (W1_1, b1_1, W2_1, b2_1, g_1, be_1),
        (W1_2, b1_2, W2_2, b2_2, g_2, be_2),
    ]
    h = x
    hs = []
    for (w1, b1, w2, b2, g, be) in params:
        agg2 = _seg_sum_sc(h, src, dst)
        h = _layer_tc(h, agg2, w1, b1, w2, b2, g, be)
        hs.append(h)
    return _head_tc(hs[0], hs[1], hs[2], b_row, Wp1, bp1, Wp2, bp2)
